# baseline (speedup 1.0000x reference)
import jax
import jax.numpy as jnp
from jax.experimental import pallas as pl

TN = 1024


def _chamfer_kernel(x_ref, yt_ref, rowmin_ref, colmin_ref):
    i = pl.program_id(1)
    xb = x_ref[0]
    ytb = yt_ref[0]
    xsq = jnp.sum(xb * xb, axis=1, keepdims=True)
    ysq = jnp.sum(ytb * ytb, axis=0, keepdims=True)
    d = xsq + ysq - 2.0 * jnp.dot(
        xb, ytb, preferred_element_type=jnp.float32,
        precision=jax.lax.Precision.HIGHEST)
    rowmin_ref[0, 0] = jnp.min(d, axis=1)
    cm = jnp.min(d, axis=0)

    @pl.when(i == 0)
    def _():
        colmin_ref[0, 0] = cm

    @pl.when(i != 0)
    def _():
        colmin_ref[0, 0] = jnp.minimum(colmin_ref[0, 0], cm)


def kernel(x, y):
    B, N, _ = x.shape
    M = y.shape[1]
    yt = jnp.transpose(y, (0, 2, 1))
    grid = (B, N // TN)
    rowmin, colmin = pl.pallas_call(
        _chamfer_kernel,
        grid=grid,
        in_specs=[
            pl.BlockSpec((1, TN, 3), lambda b, i: (b, i, 0)),
            pl.BlockSpec((1, 3, M), lambda b, i: (b, 0, 0)),
        ],
        out_specs=[
            pl.BlockSpec((1, 1, TN), lambda b, i: (b, 0, i)),
            pl.BlockSpec((1, 1, M), lambda b, i: (b, 0, 0)),
        ],
        out_shape=[
            jax.ShapeDtypeStruct((B, 1, N), jnp.float32),
            jax.ShapeDtypeStruct((B, 1, M), jnp.float32),
        ],
    )(x, yt)
    return jnp.mean(rowmin) + jnp.mean(colmin)


# speedup vs baseline: 1.0157x; 1.0157x over previous
import jax
import jax.numpy as jnp
from jax.experimental import pallas as pl

TN = 1024


def _chamfer_kernel(x_ref, yt_ref, rowmin_ref, colmin_ref):
    i = pl.program_id(1)
    xb = x_ref[0]
    ytb = yt_ref[0]
    xsq = jnp.sum(xb * xb, axis=1, keepdims=True)
    ysq = jnp.sum(ytb * ytb, axis=0, keepdims=True)
    a = jnp.concatenate([xb * -2.0, xsq, jnp.ones_like(xsq)], axis=1)
    bt = jnp.concatenate([ytb, jnp.ones_like(ysq), ysq], axis=0)
    d = jnp.dot(a, bt, preferred_element_type=jnp.float32,
                precision=jax.lax.Precision.HIGHEST)
    rowmin_ref[0, 0] = jnp.min(d, axis=1)
    cm = jnp.min(d, axis=0)

    @pl.when(i == 0)
    def _():
        colmin_ref[0, 0] = cm

    @pl.when(i != 0)
    def _():
        colmin_ref[0, 0] = jnp.minimum(colmin_ref[0, 0], cm)


def kernel(x, y):
    B, N, _ = x.shape
    M = y.shape[1]
    yt = jnp.transpose(y, (0, 2, 1))
    grid = (B, N // TN)
    rowmin, colmin = pl.pallas_call(
        _chamfer_kernel,
        grid=grid,
        in_specs=[
            pl.BlockSpec((1, TN, 3), lambda b, i: (b, i, 0)),
            pl.BlockSpec((1, 3, M), lambda b, i: (b, 0, 0)),
        ],
        out_specs=[
            pl.BlockSpec((1, 1, TN), lambda b, i: (b, 0, i)),
            pl.BlockSpec((1, 1, M), lambda b, i: (b, 0, 0)),
        ],
        out_shape=[
            jax.ShapeDtypeStruct((B, 1, N), jnp.float32),
            jax.ShapeDtypeStruct((B, 1, M), jnp.float32),
        ],
    )(x, yt)
    return jnp.mean(rowmin) + jnp.mean(colmin)


# speedup vs baseline: 1.7855x; 1.7579x over previous
import jax
import jax.numpy as jnp
from jax.experimental import pallas as pl

TN = 1024


def _chamfer_kernel(x_ref, yt_ref, rowmin_ref, colmin_ref):
    i = pl.program_id(1)
    xb = x_ref[0]
    ytb = yt_ref[0]
    xsq = jnp.sum(xb * xb, axis=1, keepdims=True)
    ysq = jnp.sum(ytb * ytb, axis=0, keepdims=True)
    a = jnp.concatenate([xb * -2.0, xsq, jnp.ones_like(xsq)], axis=1)
    bt = jnp.concatenate([ytb, jnp.ones_like(ysq), ysq], axis=0)
    ah = a.astype(jnp.bfloat16)
    al = (a - ah.astype(jnp.float32)).astype(jnp.bfloat16)
    bh = bt.astype(jnp.bfloat16)
    bl = (bt - bh.astype(jnp.float32)).astype(jnp.bfloat16)

    def _dot(p, q):
        return jnp.dot(p, q, preferred_element_type=jnp.float32)

    d = _dot(ah, bh) + _dot(ah, bl) + _dot(al, bh)
    rowmin_ref[0, 0] = jnp.min(d, axis=1)
    cm = jnp.min(d, axis=0)

    @pl.when(i == 0)
    def _():
        colmin_ref[0, 0] = cm

    @pl.when(i != 0)
    def _():
        colmin_ref[0, 0] = jnp.minimum(colmin_ref[0, 0], cm)


def kernel(x, y):
    B, N, _ = x.shape
    M = y.shape[1]
    yt = jnp.transpose(y, (0, 2, 1))
    grid = (B, N // TN)
    rowmin, colmin = pl.pallas_call(
        _chamfer_kernel,
        grid=grid,
        in_specs=[
            pl.BlockSpec((1, TN, 3), lambda b, i: (b, i, 0)),
            pl.BlockSpec((1, 3, M), lambda b, i: (b, 0, 0)),
        ],
        out_specs=[
            pl.BlockSpec((1, 1, TN), lambda b, i: (b, 0, i)),
            pl.BlockSpec((1, 1, M), lambda b, i: (b, 0, 0)),
        ],
        out_shape=[
            jax.ShapeDtypeStruct((B, 1, N), jnp.float32),
            jax.ShapeDtypeStruct((B, 1, M), jnp.float32),
        ],
    )(x, yt)
    return jnp.mean(rowmin) + jnp.mean(colmin)


# speedup vs baseline: 3.6936x; 2.0687x over previous
import jax
import jax.numpy as jnp
from jax.experimental import pallas as pl

TN = 1024


def _chamfer_kernel(x_ref, yt_ref, rowmin_ref, colmin_ref):
    i = pl.program_id(1)
    xb = x_ref[0]
    ytb = yt_ref[0]
    xsq = jnp.sum(xb * xb, axis=1, keepdims=True)
    ysq = jnp.sum(ytb * ytb, axis=0, keepdims=True)
    a = jnp.concatenate([xb * -2.0, xsq, jnp.ones_like(xsq)], axis=1)
    bt = jnp.concatenate([ytb, jnp.ones_like(ysq), ysq], axis=0)
    ah = a.astype(jnp.bfloat16)
    al = (a - ah.astype(jnp.float32)).astype(jnp.bfloat16)
    bh = bt.astype(jnp.bfloat16)
    bl = (bt - bh.astype(jnp.float32)).astype(jnp.bfloat16)
    a2 = jnp.concatenate([ah, al], axis=1)
    b2 = jnp.concatenate([bh, bl], axis=0)
    d = jnp.dot(a2, b2, preferred_element_type=jnp.float32)
    rowmin_ref[0, 0] = jnp.min(d, axis=1)
    cm = jnp.min(d, axis=0)

    @pl.when(i == 0)
    def _():
        colmin_ref[0, 0] = cm

    @pl.when(i != 0)
    def _():
        colmin_ref[0, 0] = jnp.minimum(colmin_ref[0, 0], cm)


def kernel(x, y):
    B, N, _ = x.shape
    M = y.shape[1]
    yt = jnp.transpose(y, (0, 2, 1))
    grid = (B, N // TN)
    rowmin, colmin = pl.pallas_call(
        _chamfer_kernel,
        grid=grid,
        in_specs=[
            pl.BlockSpec((1, TN, 3), lambda b, i: (b, i, 0)),
            pl.BlockSpec((1, 3, M), lambda b, i: (b, 0, 0)),
        ],
        out_specs=[
            pl.BlockSpec((1, 1, TN), lambda b, i: (b, 0, i)),
            pl.BlockSpec((1, 1, M), lambda b, i: (b, 0, 0)),
        ],
        out_shape=[
            jax.ShapeDtypeStruct((B, 1, N), jnp.float32),
            jax.ShapeDtypeStruct((B, 1, M), jnp.float32),
        ],
    )(x, yt)
    return jnp.mean(rowmin) + jnp.mean(colmin)
